# baseline (speedup 1.0000x reference)
import functools

import jax
import jax.numpy as jnp
from jax import lax
from jax.experimental import pallas as pl
from jax.experimental.pallas import tpu as pltpu
from jax.experimental.pallas import tpu_sc as plsc

B, T, D = 4096, 200, 128
OUT_LEN = 50
L = 16
NC, NS = 2, 16
NW = NC * NS
BPW = B // NW
B1 = B // 2
B2 = B - B1
BPW2 = B2 // NW
NB = 256

_mesh = plsc.VectorSubcoreMesh(core_axis_name="c", subcore_axis_name="s")


@functools.partial(
    pl.kernel,
    mesh=_mesh,
    out_type=jax.ShapeDtypeStruct((B, D), jnp.float32),
    scratch_types=[
        pltpu.VMEM((BPW,), jnp.int32),
        pltpu.VMEM((BPW,), jnp.int32),
        pltpu.VMEM((BPW, D), jnp.float32),
        pltpu.SemaphoreType.DMA,
    ],
)
def _gather_last(x_hbm, sl_hbm, out_hbm, sl_v, idx_v, rows_v, gsem):
    wid = lax.axis_index("s") * NC + lax.axis_index("c")
    base = wid * BPW
    pltpu.sync_copy(sl_hbm.at[pl.ds(base, BPW)], sl_v)
    for i in range(BPW // L):
        s = sl_v[pl.ds(i * L, L)]
        t = jnp.where(s == 0, T - 1, s - 1)
        row = (base + i * L) + lax.iota(jnp.int32, L)
        idx_v[pl.ds(i * L, L)] = row * T + t
    pltpu.async_copy(x_hbm.at[idx_v], rows_v, gsem).wait()
    pltpu.sync_copy(rows_v, out_hbm.at[pl.ds(base, BPW)])


@functools.partial(
    pl.kernel,
    mesh=_mesh,
    out_type=jax.ShapeDtypeStruct((B2, OUT_LEN, D), jnp.float32),
    scratch_types=[
        pltpu.VMEM((BPW2, D), jnp.float32),
        pltpu.SemaphoreType.DMA,
        pltpu.SemaphoreType.DMA,
    ],
)
def _sc_tile(g_hbm, out_hbm, rows_v, lsem, wsem):
    wid = lax.axis_index("s") * NC + lax.axis_index("c")
    base = wid * BPW2
    pltpu.async_copy(g_hbm.at[pl.ds(B1 + base, BPW2)], rows_v, lsem).wait()
    copies = [
        pltpu.async_copy(rows_v, out_hbm.at[pl.ds(base, BPW2), r], wsem)
        for r in range(OUT_LEN)
    ]
    for c in copies:
        c.wait()


def _tile_body(g_ref, out_ref):
    g = g_ref[...]
    out_ref[...] = jnp.broadcast_to(g[:, None, :], (NB, OUT_LEN, D))


_tc_tile = pl.pallas_call(
    _tile_body,
    grid=(B1 // NB,),
    in_specs=[pl.BlockSpec((NB, D), lambda i: (i, 0))],
    out_specs=pl.BlockSpec((NB, OUT_LEN, D), lambda i: (i, 0, 0)),
    out_shape=jax.ShapeDtypeStruct((B1, OUT_LEN, D), jnp.float32),
)


def kernel(x, seq_len, out_len):
    del out_len
    g = _gather_last(x.reshape(B * T, D), seq_len.astype(jnp.int32))
    sc_half = _sc_tile(g)
    tc_half = _tc_tile(g)
    return jnp.concatenate([tc_half, sc_half], axis=0)


# speedup vs baseline: 1.7689x; 1.7689x over previous
import functools

import jax
import jax.numpy as jnp
from jax import lax
from jax.experimental import pallas as pl
from jax.experimental.pallas import tpu as pltpu
from jax.experimental.pallas import tpu_sc as plsc

B, T, D = 4096, 200, 128
OUT_LEN = 50
L = 16
NC, NS = 2, 16
NW = NC * NS
BPW = B // NW
HB = 64

_mesh = plsc.VectorSubcoreMesh(core_axis_name="c", subcore_axis_name="s")


@functools.partial(
    pl.kernel,
    mesh=_mesh,
    out_type=jax.ShapeDtypeStruct((B, OUT_LEN, D), jnp.float32),
    scratch_types=[
        pltpu.VMEM((BPW,), jnp.int32),
        pltpu.VMEM((BPW,), jnp.int32),
        pltpu.VMEM((BPW, D), jnp.float32),
        pltpu.SemaphoreType.DMA,
        pltpu.SemaphoreType.DMA,
    ],
)
def _tile_seq_last(x_hbm, sl_hbm, out_hbm, sl_v, idx_v, rows_v, gsem, wsem):
    wid = lax.axis_index("s") * NC + lax.axis_index("c")
    base = wid * BPW

    pltpu.sync_copy(sl_hbm.at[pl.ds(base, BPW)], sl_v)
    for i in range(BPW // L):
        s = sl_v[pl.ds(i * L, L)]
        t = jnp.where(s == 0, T - 1, s - 1)
        row = (base + i * L) + lax.iota(jnp.int32, L)
        idx_v[pl.ds(i * L, L)] = row * T + t

    pltpu.async_copy(x_hbm.at[idx_v], rows_v, gsem).wait()

    copies = []
    for h in range(BPW // HB):
        for r in range(OUT_LEN):
            copies.append(pltpu.async_copy(
                rows_v.at[pl.ds(h * HB, HB)],
                out_hbm.at[pl.ds(base + h * HB, HB), r],
                wsem))
    for c in copies:
        c.wait()


def kernel(x, seq_len, out_len):
    del out_len
    return _tile_seq_last(x.reshape(B * T, D), seq_len.astype(jnp.int32))


# speedup vs baseline: 1.7829x; 1.0079x over previous
import functools

import jax
import jax.numpy as jnp
from jax import lax
from jax.experimental import pallas as pl
from jax.experimental.pallas import tpu as pltpu
from jax.experimental.pallas import tpu_sc as plsc

B, T, D = 4096, 200, 128
OUT_LEN = 50
L = 16
NC, NS = 2, 16
NW = NC * NS
BPW = B // NW
HB = 64

_mesh = plsc.VectorSubcoreMesh(core_axis_name="c", subcore_axis_name="s")


@functools.partial(
    pl.kernel,
    mesh=_mesh,
    compiler_params=pltpu.CompilerParams(use_tc_tiling_on_sc=True),
    out_type=jax.ShapeDtypeStruct((B, OUT_LEN, D), jnp.float32),
    scratch_types=[
        pltpu.VMEM((BPW,), jnp.int32),
        pltpu.VMEM((BPW,), jnp.int32),
        pltpu.VMEM((BPW, D), jnp.float32),
        pltpu.SemaphoreType.DMA,
        pltpu.SemaphoreType.DMA,
    ],
)
def _tile_seq_last(x_hbm, sl_hbm, out_hbm, sl_v, idx_v, rows_v, gsem, wsem):
    wid = lax.axis_index("s") * NC + lax.axis_index("c")
    base = wid * BPW

    pltpu.sync_copy(sl_hbm.at[pl.ds(base, BPW)], sl_v)
    for i in range(BPW // L):
        s = sl_v[pl.ds(i * L, L)]
        t = jnp.where(s == 0, T - 1, s - 1)
        row = (base + i * L) + lax.iota(jnp.int32, L)
        idx_v[pl.ds(i * L, L)] = row * T + t

    pltpu.async_copy(x_hbm.at[idx_v], rows_v, gsem).wait()

    copies = []
    for h in range(BPW // HB):
        for r in range(OUT_LEN):
            copies.append(pltpu.async_copy(
                rows_v.at[pl.ds(h * HB, HB)],
                out_hbm.at[pl.ds(base + h * HB, HB), r],
                wsem))
    for c in copies:
        c.wait()


def kernel(x, seq_len, out_len):
    del out_len
    return _tile_seq_last(x.reshape(B * T, D), seq_len.astype(jnp.int32))
